# baseline (speedup 1.0000x reference)
import functools

import jax
import jax.numpy as jnp
from jax.experimental import pallas as pl
from jax.experimental.pallas import tpu as pltpu

NUM_EXPERTS = 8
GROUP_SIZE = 2
HIDDEN = 1024
FF = 2730
FF_PAD = 2816
FT = 1408
NF = FF_PAD // FT


def _ffn_body(tok_ref, wup_ref, bup_ref, wdn_ref, bdn_ref, g_ref, out_ref,
              acc_ref, *, k):
    f = pl.program_id(1)

    @pl.when(f == 0)
    def _init():
        acc_ref[...] = jnp.broadcast_to(bdn_ref[0], (k, HIDDEN))

    tok = tok_ref[0]
    up = jnp.dot(tok, wup_ref[0], preferred_element_type=jnp.float32)
    up = up + bup_ref[0]
    a = (up * jax.nn.sigmoid(up)).astype(jnp.bfloat16)
    acc_ref[...] += jnp.dot(a, wdn_ref[0], preferred_element_type=jnp.float32)

    @pl.when(f == NF - 1)
    def _finish():
        g = g_ref[0][:, :1]
        out_ref[...] = (acc_ref[...] * g)[None]


def _ffn(tokens, W_up, b_up, W_down, b_down, G, k):
    E = NUM_EXPERTS
    grid = (E, NF)
    return pl.pallas_call(
        functools.partial(_ffn_body, k=k),
        grid=grid,
        in_specs=[
            pl.BlockSpec((1, k, HIDDEN), lambda e, f: (e, 0, 0)),
            pl.BlockSpec((1, HIDDEN, FT), lambda e, f: (e, 0, f)),
            pl.BlockSpec((1, 1, FT), lambda e, f: (e, 0, f)),
            pl.BlockSpec((1, FT, HIDDEN), lambda e, f: (e // GROUP_SIZE, f, 0)),
            pl.BlockSpec((1, 1, HIDDEN), lambda e, f: (e // GROUP_SIZE, 0, 0)),
            pl.BlockSpec((1, k, 128), lambda e, f: (e, 0, 0)),
        ],
        out_specs=pl.BlockSpec((1, k, HIDDEN), lambda e, f: (e, 0, 0)),
        out_shape=jax.ShapeDtypeStruct((E, k, HIDDEN), jnp.float32),
        scratch_shapes=[pltpu.VMEM((k, HIDDEN), jnp.float32)],
        compiler_params=pltpu.CompilerParams(
            dimension_semantics=("arbitrary", "arbitrary")),
    )(tokens, W_up, b_up, W_down, b_down, G)


def kernel(x, routing_logits, batch_size, seq_len, W_up, b_up, W_down, b_down):
    bs, hidden = x.shape
    k = min(bs // NUM_EXPERTS, bs)

    S = jax.nn.softmax(routing_logits, axis=-1)
    G_t, idx_t = jax.lax.top_k(S.T, k)
    tokens = jnp.take(x, idx_t, axis=0)

    tokens_bf = tokens.astype(jnp.bfloat16)
    wup_bf = jnp.pad(W_up, ((0, 0), (0, 0), (0, FF_PAD - FF))).astype(jnp.bfloat16)
    bup_p = jnp.pad(b_up, ((0, 0), (0, FF_PAD - FF)))[:, None, :]
    bdn_r = b_down[:, None, :]
    wdn_bf = jnp.pad(W_down, ((0, 0), (0, FF_PAD - FF), (0, 0))).astype(jnp.bfloat16)
    Gb = jnp.broadcast_to(G_t[:, :, None], (NUM_EXPERTS, k, 128))

    weighted = _ffn(tokens_bf, wup_bf, bup_p, wdn_bf, bdn_r, Gb, k)

    y = jnp.zeros((bs, hidden), dtype=x.dtype).at[idx_t.reshape(-1)].add(
        weighted.reshape(-1, hidden))
    return y
